# baseline (speedup 1.0000x reference)
import jax
import jax.numpy as jnp
from jax import lax
from jax.experimental import pallas as pl
from jax.experimental.pallas import tpu as pltpu

_B, _N, _E, _A = 8, 4096, 128, 64
_RN, _RC = 32, 128
_PAD = -2.0


def _body(x_ref, m_ref, bc_ref, wq_ref, wk_ref, wv_ref, g_ref, w_ref,
          gam_ref, bet_ref, outs_ref, sel_ref, wts_ref, bout_ref):
    q = jnp.dot(bc_ref[0], wq_ref[...])
    qT = q.reshape(_E, 1)
    scale = 1.0 / jnp.sqrt(jnp.float32(_E))
    k_all = jnp.dot(x_ref[...].reshape(_B * _N, _E), wk_ref[...])
    lg = jnp.dot(k_all, qT).reshape(_B, _RN, _RC) * scale

    m01 = jnp.where(m_ref[...] == _PAD, 0.0, 1.0)
    lga = lg + (1.0 - m01) * (-1e9)
    mx = jnp.max(jnp.max(lga, axis=2, keepdims=True), axis=1, keepdims=True)
    ex = jnp.exp(lga - mx)
    den = jnp.sum(jnp.sum(ex, axis=2, keepdims=True), axis=1, keepdims=True)
    att = ex / den * m01

    x5 = x_ref[...].reshape(_B, _RN, _RC, _E)
    att4 = att.reshape(_B, _RN, _RC, 1)
    s = jnp.sum(jnp.sum(att4 * x5, axis=2), axis=1)
    bout_ref[...] = jnp.dot(s, wv_ref[...],
                            precision=lax.Precision.HIGHEST)

    idx3 = (lax.broadcasted_iota(jnp.int32, (1, _RN, _RC), 1) * _RC
            + lax.broadcasted_iota(jnp.int32, (1, _RN, _RC), 2))
    lane_a = lax.broadcasted_iota(jnp.int32, (1, 1, _A), 2)

    def gbody(i, st):
        scores, sel3, wv3 = st
        mval = jnp.max(jnp.max(scores, axis=2, keepdims=True),
                       axis=1, keepdims=True)
        cidx = jnp.where(scores == mval, idx3, jnp.int32(1 << 30))
        cand = jnp.min(jnp.min(cidx, axis=2, keepdims=True),
                       axis=1, keepdims=True)
        sel3 = jnp.where(lane_a == i, cand, sel3)
        wv3 = jnp.where(lane_a == i, mval, wv3)
        scores = jnp.where(jnp.abs(idx3 - cand) <= 3, -1.0, scores)
        return scores, sel3, wv3

    init = (att,
            jnp.zeros((_B, 1, _A), jnp.int32),
            jnp.zeros((_B, 1, _A), jnp.float32))
    _, sel3, wv3 = lax.fori_loop(0, _A, gbody, init)

    sel_f = sel3.astype(jnp.float32)
    sel_c = sel_f.reshape(_B, _A, 1)
    rank_r = jnp.sum((sel_c < sel_f).astype(jnp.float32), axis=1,
                     keepdims=True)
    rank_c = rank_r.reshape(_B, _A, 1)
    io_r = lane_a.astype(jnp.float32)
    Q = (rank_c == io_r).astype(jnp.float32)
    sel_sorted_r = jnp.sum(Q * sel_c, axis=1, keepdims=True)
    wts_sorted_r = jnp.sum(Q * wv3.reshape(_B, _A, 1), axis=1,
                           keepdims=True)
    sel_sorted_c = sel_sorted_r.reshape(_B, _A, 1)
    wts_sorted_c = wts_sorted_r.reshape(_B, _A, 1)

    sel_ref[...] = sel_sorted_r.astype(jnp.int32)
    wts_ref[...] = wts_sorted_r

    idx_row = lax.broadcasted_iota(jnp.int32, (1, 1, _N), 2)
    onehot = (sel_sorted_c.astype(jnp.int32) == idx_row).astype(jnp.float32)
    ra = (lax.broadcasted_iota(jnp.int32, (1, _E), 1).astype(jnp.float32)
          / jnp.float32(_E))
    denom = jnp.exp(jnp.log(jnp.float32(40.0)) * ra)
    for b in range(_B):
        rows_s = jnp.dot(onehot[b], x_ref[b],
                         precision=lax.Precision.HIGHEST)
        pos = sel_sorted_c[b]
        o = rows_s + jnp.sin(pos / denom)
        o3 = o.reshape(_A, _E, 1)
        ow = jnp.sum(o3 * w_ref[...], axis=1)
        og = jnp.sum(o3 * g_ref[...], axis=1)
        out = ow * jax.nn.sigmoid(og) * wts_sorted_c[b]
        mu = jnp.mean(out, axis=1, keepdims=True)
        var = jnp.mean((out - mu) ** 2, axis=1, keepdims=True)
        outs_ref[b] = ((out - mu) * lax.rsqrt(var + 1e-3) * gam_ref[...]
                       + bet_ref[...])


def kernel(x, mask, barcode, Wq, Wk, Wv, g, w, gamma, beta):
    B, N, E = x.shape
    A = g.shape[0]
    mask_r = mask.reshape(B, N // _RC, _RC)
    w_r = w.reshape(A, E, E)
    gam = gamma.reshape(1, E)
    bet = beta.reshape(1, E)
    outs, sel3, wts3, bout = pl.pallas_call(
        _body,
        grid=(1,),
        in_specs=[
            pl.BlockSpec((B, N, E), lambda i: (0, 0, 0)),
            pl.BlockSpec((B, N // _RC, _RC), lambda i: (0, 0, 0)),
            pl.BlockSpec((1, 1, E), lambda i: (0, 0, 0)),
            pl.BlockSpec((E, E), lambda i: (0, 0)),
            pl.BlockSpec((E, E), lambda i: (0, 0)),
            pl.BlockSpec((E, E), lambda i: (0, 0)),
            pl.BlockSpec((A, E, E), lambda i: (0, 0, 0)),
            pl.BlockSpec((A, E, E), lambda i: (0, 0, 0)),
            pl.BlockSpec((1, E), lambda i: (0, 0)),
            pl.BlockSpec((1, E), lambda i: (0, 0)),
        ],
        out_specs=(
            pl.BlockSpec((B, A, E), lambda i: (0, 0, 0)),
            pl.BlockSpec((B, 1, A), lambda i: (0, 0, 0)),
            pl.BlockSpec((B, 1, A), lambda i: (0, 0, 0)),
            pl.BlockSpec((B, E), lambda i: (0, 0)),
        ),
        out_shape=(
            jax.ShapeDtypeStruct((B, A, E), jnp.float32),
            jax.ShapeDtypeStruct((B, 1, A), jnp.int32),
            jax.ShapeDtypeStruct((B, 1, A), jnp.float32),
            jax.ShapeDtypeStruct((B, E), jnp.float32),
        ),
        compiler_params=pltpu.CompilerParams(
            dimension_semantics=("arbitrary",)),
    )(x, mask_r, barcode, Wq, Wk, Wv, g, w_r, gam, bet)
    return outs, sel3.reshape(B, A), wts3.reshape(B, A), bout


# speedup vs baseline: 1.0005x; 1.0005x over previous
import jax
import jax.numpy as jnp
from jax import lax
from jax.experimental import pallas as pl
from jax.experimental.pallas import tpu as pltpu

_B, _N, _E, _A = 8, 4096, 128, 64
_RN, _RC = 32, 128
_PAD = -2.0


def _body(x_ref, m_ref, bc_ref, wq_ref, wk_ref, wv_ref, g_ref, w_ref,
          gam_ref, bet_ref, outs_ref, sel_ref, wts_ref, bout_ref):
    q = jnp.dot(bc_ref[0], wq_ref[...])
    qT = q.reshape(_E, 1)
    scale = 1.0 / jnp.sqrt(jnp.float32(_E))
    lgs = []
    for b in range(_B):
        k_b = jnp.dot(x_ref[b], wk_ref[...])
        lgs.append(jnp.dot(k_b, qT).reshape(1, _RN, _RC))
    lg = jnp.concatenate(lgs, axis=0) * scale

    m01 = jnp.where(m_ref[...] == _PAD, 0.0, 1.0)
    lga = lg + (1.0 - m01) * (-1e9)
    mx = jnp.max(jnp.max(lga, axis=2, keepdims=True), axis=1, keepdims=True)
    ex = jnp.exp(lga - mx)
    den = jnp.sum(jnp.sum(ex, axis=2, keepdims=True), axis=1, keepdims=True)
    att = ex / den * m01

    x5 = x_ref[...].reshape(_B, _RN, _RC, _E)
    att4 = att.reshape(_B, _RN, _RC, 1)
    s = jnp.sum(jnp.sum(att4 * x5, axis=2), axis=1)
    bout_ref[...] = jnp.dot(s, wv_ref[...],
                            precision=lax.Precision.HIGHEST)

    idx3 = (lax.broadcasted_iota(jnp.int32, (1, _RN, _RC), 1) * _RC
            + lax.broadcasted_iota(jnp.int32, (1, _RN, _RC), 2))
    lane_a = lax.broadcasted_iota(jnp.int32, (1, 1, _A), 2)

    def gbody(i, st):
        scores, sel3, wv3 = st
        mval = jnp.max(jnp.max(scores, axis=2, keepdims=True),
                       axis=1, keepdims=True)
        cidx = jnp.where(scores == mval, idx3, jnp.int32(1 << 30))
        cand = jnp.min(jnp.min(cidx, axis=2, keepdims=True),
                       axis=1, keepdims=True)
        sel3 = jnp.where(lane_a == i, cand, sel3)
        wv3 = jnp.where(lane_a == i, mval, wv3)
        scores = jnp.where(jnp.abs(idx3 - cand) <= 3, -1.0, scores)
        return scores, sel3, wv3

    init = (att,
            jnp.zeros((_B, 1, _A), jnp.int32),
            jnp.zeros((_B, 1, _A), jnp.float32))
    _, sel3, wv3 = lax.fori_loop(0, _A, gbody, init)

    sel_f = sel3.astype(jnp.float32)
    sel_c = sel_f.reshape(_B, _A, 1)
    rank_r = jnp.sum((sel_c < sel_f).astype(jnp.float32), axis=1,
                     keepdims=True)
    rank_c = rank_r.reshape(_B, _A, 1)
    io_r = lane_a.astype(jnp.float32)
    Q = (rank_c == io_r).astype(jnp.float32)
    sel_sorted_r = jnp.sum(Q * sel_c, axis=1, keepdims=True)
    wts_sorted_r = jnp.sum(Q * wv3.reshape(_B, _A, 1), axis=1,
                           keepdims=True)
    sel_sorted_c = sel_sorted_r.reshape(_B, _A, 1)
    wts_sorted_c = wts_sorted_r.reshape(_B, _A, 1)

    sel_ref[...] = sel_sorted_r.astype(jnp.int32)
    wts_ref[...] = wts_sorted_r

    idx_row = lax.broadcasted_iota(jnp.int32, (1, 1, _N), 2)
    onehot = (sel_sorted_c.astype(jnp.int32) == idx_row).astype(jnp.float32)
    ra = (lax.broadcasted_iota(jnp.int32, (1, _E), 1).astype(jnp.float32)
          / jnp.float32(_E))
    denom = jnp.exp(jnp.log(jnp.float32(40.0)) * ra)
    for b in range(_B):
        rows_s = jnp.dot(onehot[b], x_ref[b],
                         precision=lax.Precision.HIGHEST)
        pos = sel_sorted_c[b]
        o = rows_s + jnp.sin(pos / denom)
        o3 = o.reshape(_A, _E, 1)
        ow = jnp.sum(o3 * w_ref[...], axis=1)
        og = jnp.sum(o3 * g_ref[...], axis=1)
        out = ow * jax.nn.sigmoid(og) * wts_sorted_c[b]
        mu = jnp.mean(out, axis=1, keepdims=True)
        var = jnp.mean((out - mu) ** 2, axis=1, keepdims=True)
        outs_ref[b] = ((out - mu) * lax.rsqrt(var + 1e-3) * gam_ref[...]
                       + bet_ref[...])


def kernel(x, mask, barcode, Wq, Wk, Wv, g, w, gamma, beta):
    B, N, E = x.shape
    A = g.shape[0]
    mask_r = mask.reshape(B, N // _RC, _RC)
    w_r = w.reshape(A, E, E)
    gam = gamma.reshape(1, E)
    bet = beta.reshape(1, E)
    outs, sel3, wts3, bout = pl.pallas_call(
        _body,
        grid=(1,),
        in_specs=[
            pl.BlockSpec((B, N, E), lambda i: (0, 0, 0)),
            pl.BlockSpec((B, N // _RC, _RC), lambda i: (0, 0, 0)),
            pl.BlockSpec((1, 1, E), lambda i: (0, 0, 0)),
            pl.BlockSpec((E, E), lambda i: (0, 0)),
            pl.BlockSpec((E, E), lambda i: (0, 0)),
            pl.BlockSpec((E, E), lambda i: (0, 0)),
            pl.BlockSpec((A, E, E), lambda i: (0, 0, 0)),
            pl.BlockSpec((A, E, E), lambda i: (0, 0, 0)),
            pl.BlockSpec((1, E), lambda i: (0, 0)),
            pl.BlockSpec((1, E), lambda i: (0, 0)),
        ],
        out_specs=(
            pl.BlockSpec((B, A, E), lambda i: (0, 0, 0)),
            pl.BlockSpec((B, 1, A), lambda i: (0, 0, 0)),
            pl.BlockSpec((B, 1, A), lambda i: (0, 0, 0)),
            pl.BlockSpec((B, E), lambda i: (0, 0)),
        ),
        out_shape=(
            jax.ShapeDtypeStruct((B, A, E), jnp.float32),
            jax.ShapeDtypeStruct((B, 1, A), jnp.int32),
            jax.ShapeDtypeStruct((B, 1, A), jnp.float32),
            jax.ShapeDtypeStruct((B, E), jnp.float32),
        ),
        compiler_params=pltpu.CompilerParams(
            dimension_semantics=("arbitrary",)),
    )(x, mask_r, barcode, Wq, Wk, Wv, g, w_r, gam, bet)
    return outs, sel3.reshape(B, A), wts3.reshape(B, A), bout


# speedup vs baseline: 1.3192x; 1.3186x over previous
import jax
import jax.numpy as jnp
from jax import lax
from jax.experimental import pallas as pl
from jax.experimental.pallas import tpu as pltpu

_B, _N, _E, _A = 8, 4096, 128, 64
_RN, _RC = 32, 128
_PAD = -2.0


def _body(x_ref, m_ref, bc_ref, wq_ref, wk_ref, wv_ref, g_ref, w_ref,
          gam_ref, bet_ref, outs_ref, sel_ref, wts_ref, bout_ref, att_s):
    q = jnp.dot(bc_ref[0], wq_ref[...])
    qT = q.reshape(_E, 1)
    scale = 1.0 / jnp.sqrt(jnp.float32(_E))
    for b in range(_B):
        k_b = jnp.dot(x_ref[b], wk_ref[...])
        att_s[b] = jnp.dot(k_b, qT).reshape(_RN, _RC) * scale

    m01 = jnp.where(m_ref[...] == _PAD, 0.0, 1.0)
    lga = att_s[...] + (1.0 - m01) * (-1e9)
    mx = jnp.max(jnp.max(lga, axis=2, keepdims=True), axis=1, keepdims=True)
    ex = jnp.exp(lga - mx)
    den = jnp.sum(jnp.sum(ex, axis=2, keepdims=True), axis=1, keepdims=True)
    att = ex / den * m01

    x5 = x_ref[...].reshape(_B, _RN, _RC, _E)
    att4 = att.reshape(_B, _RN, _RC, 1)
    s = jnp.sum(jnp.sum(att4 * x5, axis=2), axis=1)
    bout_ref[...] = jnp.dot(s, wv_ref[...],
                            precision=lax.Precision.HIGHEST)

    idx3 = (lax.broadcasted_iota(jnp.int32, (1, _RN, _RC), 1) * _RC
            + lax.broadcasted_iota(jnp.int32, (1, _RN, _RC), 2))
    lane_a = lax.broadcasted_iota(jnp.int32, (1, 1, _A), 2)

    def gbody(i, st):
        scores, sel3, wv3 = st
        mval = jnp.max(jnp.max(scores, axis=2, keepdims=True),
                       axis=1, keepdims=True)
        cidx = jnp.where(scores == mval, idx3, jnp.int32(1 << 30))
        cand = jnp.min(jnp.min(cidx, axis=2, keepdims=True),
                       axis=1, keepdims=True)
        sel3 = jnp.where(lane_a == i, cand, sel3)
        wv3 = jnp.where(lane_a == i, mval, wv3)
        scores = jnp.where(jnp.abs(idx3 - cand) <= 3, -1.0, scores)
        return scores, sel3, wv3

    init = (att,
            jnp.zeros((_B, 1, _A), jnp.int32),
            jnp.zeros((_B, 1, _A), jnp.float32))
    _, sel3, wv3 = lax.fori_loop(0, _A, gbody, init)

    sel_f = sel3.astype(jnp.float32)
    sel_c = sel_f.reshape(_B, _A, 1)
    rank_r = jnp.sum((sel_c < sel_f).astype(jnp.float32), axis=1,
                     keepdims=True)
    rank_c = rank_r.reshape(_B, _A, 1)
    io_r = lane_a.astype(jnp.float32)
    Q = (rank_c == io_r).astype(jnp.float32)
    sel_sorted_r = jnp.sum(Q * sel_c, axis=1, keepdims=True)
    wts_sorted_r = jnp.sum(Q * wv3.reshape(_B, _A, 1), axis=1,
                           keepdims=True)
    sel_sorted_c = sel_sorted_r.reshape(_B, _A, 1)
    wts_sorted_c = wts_sorted_r.reshape(_B, _A, 1)

    sel_ref[...] = sel_sorted_r.astype(jnp.int32)
    wts_ref[...] = wts_sorted_r

    idx_row = lax.broadcasted_iota(jnp.int32, (1, 1, _N), 2)
    onehot = (sel_sorted_c.astype(jnp.int32) == idx_row).astype(jnp.float32)
    ra = (lax.broadcasted_iota(jnp.int32, (1, _E), 1).astype(jnp.float32)
          / jnp.float32(_E))
    denom = jnp.exp(jnp.log(jnp.float32(40.0)) * ra)
    for b in range(_B):
        rows_s = jnp.dot(onehot[b], x_ref[b],
                         precision=lax.Precision.HIGHEST)
        pos = sel_sorted_c[b]
        o = rows_s + jnp.sin(pos / denom)
        o3 = o.reshape(_A, _E, 1)
        ow = jnp.sum(o3 * w_ref[...], axis=1)
        og = jnp.sum(o3 * g_ref[...], axis=1)
        out = ow * jax.nn.sigmoid(og) * wts_sorted_c[b]
        mu = jnp.mean(out, axis=1, keepdims=True)
        var = jnp.mean((out - mu) ** 2, axis=1, keepdims=True)
        outs_ref[b] = ((out - mu) * lax.rsqrt(var + 1e-3) * gam_ref[...]
                       + bet_ref[...])


def kernel(x, mask, barcode, Wq, Wk, Wv, g, w, gamma, beta):
    B, N, E = x.shape
    A = g.shape[0]
    mask_r = mask.reshape(B, N // _RC, _RC)
    w_r = w.reshape(A, E, E)
    gam = gamma.reshape(1, E)
    bet = beta.reshape(1, E)
    outs, sel3, wts3, bout = pl.pallas_call(
        _body,
        grid=(1,),
        in_specs=[
            pl.BlockSpec((B, N, E), lambda i: (0, 0, 0)),
            pl.BlockSpec((B, N // _RC, _RC), lambda i: (0, 0, 0)),
            pl.BlockSpec((1, 1, E), lambda i: (0, 0, 0)),
            pl.BlockSpec((E, E), lambda i: (0, 0)),
            pl.BlockSpec((E, E), lambda i: (0, 0)),
            pl.BlockSpec((E, E), lambda i: (0, 0)),
            pl.BlockSpec((A, E, E), lambda i: (0, 0, 0)),
            pl.BlockSpec((A, E, E), lambda i: (0, 0, 0)),
            pl.BlockSpec((1, E), lambda i: (0, 0)),
            pl.BlockSpec((1, E), lambda i: (0, 0)),
        ],
        out_specs=(
            pl.BlockSpec((B, A, E), lambda i: (0, 0, 0)),
            pl.BlockSpec((B, 1, A), lambda i: (0, 0, 0)),
            pl.BlockSpec((B, 1, A), lambda i: (0, 0, 0)),
            pl.BlockSpec((B, E), lambda i: (0, 0)),
        ),
        out_shape=(
            jax.ShapeDtypeStruct((B, A, E), jnp.float32),
            jax.ShapeDtypeStruct((B, 1, A), jnp.int32),
            jax.ShapeDtypeStruct((B, 1, A), jnp.float32),
            jax.ShapeDtypeStruct((B, E), jnp.float32),
        ),
        scratch_shapes=[pltpu.VMEM((_B, _RN, _RC), jnp.float32)],
        compiler_params=pltpu.CompilerParams(
            dimension_semantics=("arbitrary",)),
    )(x, mask_r, barcode, Wq, Wk, Wv, g, w_r, gam, bet)
    return outs, sel3.reshape(B, A), wts3.reshape(B, A), bout
